# Initial kernel scaffold; baseline (speedup 1.0000x reference)
import functools

import jax
import jax.numpy as jnp
from jax import lax
from jax.experimental import pallas as pl
from jax.experimental.pallas import tpu as pltpu
from jax.experimental.pallas import tpu_sc as plsc

NC = 2
NS = 16
NW = NC * NS


def _pick_chunk(epw):
    for c in range(128, 7, -8):
        if epw % c == 0:
            return c
    return None


def _sc_aggregate(h, src, dst, w, zeros):
    n, d = h.shape
    e = src.shape[0]
    epw = e // NW
    ch = _pick_chunk(epw)
    nch = epw // ch
    rpt = n // NS

    mesh = plsc.VectorSubcoreMesh(core_axis_name="c", subcore_axis_name="s")

    @functools.partial(
        pl.kernel,
        out_type=jax.ShapeDtypeStruct((NC, n, d), jnp.float32),
        mesh=mesh,
        scratch_types=[
            pltpu.VMEM((ch,), jnp.int32),
            pltpu.VMEM((ch,), jnp.int32),
            pltpu.VMEM((ch,), jnp.float32),
            pltpu.VMEM((ch, d), jnp.float32),
            pltpu.VMEM_SHARED((n, d), jnp.float32),
            pltpu.SemaphoreType.DMA,
        ],
    )
    def k(h_hbm, src_hbm, dst_hbm, w_hbm, z_hbm, out_hbm,
          srcv, dstv, wv, rows, acc, sem):
        cid = lax.axis_index("c")
        sid = lax.axis_index("s")
        wid = sid * NC + cid
        r0 = sid * rpt
        pltpu.sync_copy(z_hbm, acc.at[pl.ds(r0, rpt)])
        plsc.subcore_barrier()

        ebase = wid * epw

        def body(i, carry):
            off = pl.multiple_of(ebase + i * ch, 8)
            pltpu.sync_copy(src_hbm.at[pl.ds(off, ch)], srcv)
            pltpu.sync_copy(dst_hbm.at[pl.ds(off, ch)], dstv)
            pltpu.sync_copy(w_hbm.at[pl.ds(off, ch)], wv)
            pltpu.async_copy(h_hbm.at[srcv], rows, sem).wait()
            for ei in range(ch):
                we = wv[ei]
                for kk in range(d // 16):
                    sl = pl.ds(kk * 16, 16)
                    rows[ei, sl] = rows[ei, sl] * we
            pltpu.sync_copy(rows, acc.at[dstv], add=True)
            return carry

        lax.fori_loop(0, nch, body, 0)
        plsc.subcore_barrier()
        pltpu.sync_copy(acc.at[pl.ds(r0, rpt)], out_hbm.at[cid, pl.ds(r0, rpt)])

    return k(h, src, dst, w, zeros)


def _tc_matmul(x, w):
    m, kdim = x.shape
    nout = w.shape[1]
    nb = 8
    bm = m // nb

    def body(x_ref, w_ref, o_ref):
        o_ref[...] = jnp.dot(x_ref[...], w_ref[...],
                             preferred_element_type=jnp.float32)

    return pl.pallas_call(
        body,
        grid=(nb,),
        in_specs=[pl.BlockSpec((bm, kdim), lambda i: (i, 0)),
                  pl.BlockSpec((kdim, nout), lambda i: (0, 0))],
        out_specs=pl.BlockSpec((bm, nout), lambda i: (i, 0)),
        out_shape=jax.ShapeDtypeStruct((m, nout), jnp.float32),
    )(x, w)


def _tc_combine_relu_matmul(parts, b, w):
    _, m, hdim = parts.shape
    nout = w.shape[1]
    nb = 8
    bm = m // nb
    b2d = b.reshape(1, hdim)

    def body(p_ref, b_ref, w_ref, o_ref):
        x = jnp.maximum(p_ref[0] + p_ref[1] + b_ref[...], 0.0)
        o_ref[...] = jnp.dot(x, w_ref[...], preferred_element_type=jnp.float32)

    return pl.pallas_call(
        body,
        grid=(nb,),
        in_specs=[pl.BlockSpec((NC, bm, hdim), lambda i: (0, i, 0)),
                  pl.BlockSpec((1, hdim), lambda i: (0, 0)),
                  pl.BlockSpec((hdim, nout), lambda i: (0, 0))],
        out_specs=pl.BlockSpec((bm, nout), lambda i: (i, 0)),
        out_shape=jax.ShapeDtypeStruct((m, nout), jnp.float32),
    )(parts, b2d, w)


def _tc_combine_bias(parts, b):
    _, m, hdim = parts.shape
    nb = 8
    bm = m // nb
    b2d = b.reshape(1, hdim)

    def body(p_ref, b_ref, o_ref):
        o_ref[...] = p_ref[0] + p_ref[1] + b_ref[...]

    return pl.pallas_call(
        body,
        grid=(nb,),
        in_specs=[pl.BlockSpec((NC, bm, hdim), lambda i: (0, i, 0)),
                  pl.BlockSpec((1, hdim), lambda i: (0, 0))],
        out_specs=pl.BlockSpec((bm, hdim), lambda i: (i, 0)),
        out_shape=jax.ShapeDtypeStruct((m, hdim), jnp.float32),
    )(parts, b2d)


def kernel(edge_index, edge_weight, emb, W1, b1, W2, b2):
    src = edge_index[0]
    dst = edge_index[1]
    n, d = emb.shape
    zeros = jnp.zeros((n // NS, d), dtype=jnp.float32)

    h1 = _tc_matmul(emb, W1)
    parts1 = _sc_aggregate(h1, src, dst, edge_weight, zeros)
    h2 = _tc_combine_relu_matmul(parts1, b1, W2)
    parts2 = _sc_aggregate(h2, src, dst, edge_weight, zeros)
    return _tc_combine_bias(parts2, b2)


# speedup vs baseline: 4.1865x; 4.1865x over previous
import functools

import jax
import jax.numpy as jnp
from jax import lax
from jax.experimental import pallas as pl
from jax.experimental.pallas import tpu as pltpu
from jax.experimental.pallas import tpu_sc as plsc

NC = 2
NS = 16
NW = NC * NS


def _pick_chunk(epw):
    for c in range(128, 7, -8):
        if epw % c == 0:
            return c
    return None


def _sc_aggregate(h, src, dst, w, zeros):
    n, d = h.shape
    e = src.shape[0]
    epw = e // NW
    ch = _pick_chunk(epw)
    nch = epw // ch
    rpt = (n // NS) & ~7
    tail = n - NS * rpt

    mesh = plsc.VectorSubcoreMesh(core_axis_name="c", subcore_axis_name="s")

    @functools.partial(
        pl.kernel,
        out_type=jax.ShapeDtypeStruct((NC, n, d), jnp.float32),
        mesh=mesh,
        scratch_types=[
            pltpu.VMEM((ch,), jnp.int32),
            pltpu.VMEM((ch,), jnp.int32),
            pltpu.VMEM((ch,), jnp.float32),
            pltpu.VMEM((ch, d), jnp.float32),
            pltpu.VMEM_SHARED((n, d), jnp.float32),
            pltpu.SemaphoreType.DMA,
        ],
    )
    def k(h_hbm, src_hbm, dst_hbm, w_hbm, z_hbm, out_hbm,
          srcv, dstv, wv, rows, acc, sem):
        cid = lax.axis_index("c")
        sid = lax.axis_index("s")
        wid = sid * NC + cid
        r0 = pl.multiple_of(sid * rpt, 8)
        pltpu.sync_copy(z_hbm.at[pl.ds(0, rpt)], acc.at[pl.ds(r0, rpt)])
        if tail:
            @pl.when(sid == NS - 1)
            def _():
                pltpu.sync_copy(z_hbm.at[pl.ds(0, tail)],
                                acc.at[pl.ds(NS * rpt, tail)])
        plsc.subcore_barrier()

        ebase = wid * epw

        def body(i, carry):
            off = pl.multiple_of(ebase + i * ch, 8)
            pltpu.sync_copy(src_hbm.at[pl.ds(off, ch)], srcv)
            pltpu.sync_copy(dst_hbm.at[pl.ds(off, ch)], dstv)
            pltpu.sync_copy(w_hbm.at[pl.ds(off, ch)], wv)
            pltpu.async_copy(h_hbm.at[srcv], rows, sem).wait()
            for g in range(ch // 16):
                wvec = wv[pl.ds(g * 16, 16)]
                for j in range(16):
                    ei = g * 16 + j
                    we = wvec[j]
                    for kk in range(d // 16):
                        sl = pl.ds(kk * 16, 16)
                        rows[ei, sl] = rows[ei, sl] * we
            pltpu.sync_copy(rows, acc.at[dstv], add=True)
            return carry

        lax.fori_loop(0, nch, body, 0)
        plsc.subcore_barrier()
        pltpu.sync_copy(acc.at[pl.ds(r0, rpt)], out_hbm.at[cid, pl.ds(r0, rpt)])
        if tail:
            @pl.when(sid == NS - 1)
            def _():
                pltpu.sync_copy(acc.at[pl.ds(NS * rpt, tail)],
                                out_hbm.at[cid, pl.ds(NS * rpt, tail)])

    return k(h, src, dst, w, zeros)


def _tc_matmul(x, w):
    m, kdim = x.shape
    nout = w.shape[1]
    nb = 10
    bm = m // nb

    def body(x_ref, w_ref, o_ref):
        o_ref[...] = jnp.dot(x_ref[...], w_ref[...],
                             preferred_element_type=jnp.float32)

    return pl.pallas_call(
        body,
        grid=(nb,),
        in_specs=[pl.BlockSpec((bm, kdim), lambda i: (i, 0)),
                  pl.BlockSpec((kdim, nout), lambda i: (0, 0))],
        out_specs=pl.BlockSpec((bm, nout), lambda i: (i, 0)),
        out_shape=jax.ShapeDtypeStruct((m, nout), jnp.float32),
    )(x, w)


def _tc_combine_relu_matmul(parts, b, w):
    _, m, hdim = parts.shape
    nout = w.shape[1]
    nb = 10
    bm = m // nb
    b2d = b.reshape(1, hdim)

    def body(p_ref, b_ref, w_ref, o_ref):
        x = jnp.maximum(p_ref[0] + p_ref[1] + b_ref[...], 0.0)
        o_ref[...] = jnp.dot(x, w_ref[...], preferred_element_type=jnp.float32)

    return pl.pallas_call(
        body,
        grid=(nb,),
        in_specs=[pl.BlockSpec((NC, bm, hdim), lambda i: (0, i, 0)),
                  pl.BlockSpec((1, hdim), lambda i: (0, 0)),
                  pl.BlockSpec((hdim, nout), lambda i: (0, 0))],
        out_specs=pl.BlockSpec((bm, nout), lambda i: (i, 0)),
        out_shape=jax.ShapeDtypeStruct((m, nout), jnp.float32),
    )(parts, b2d, w)


def _tc_combine_bias(parts, b):
    _, m, hdim = parts.shape
    nb = 10
    bm = m // nb
    b2d = b.reshape(1, hdim)

    def body(p_ref, b_ref, o_ref):
        o_ref[...] = p_ref[0] + p_ref[1] + b_ref[...]

    return pl.pallas_call(
        body,
        grid=(nb,),
        in_specs=[pl.BlockSpec((NC, bm, hdim), lambda i: (0, i, 0)),
                  pl.BlockSpec((1, hdim), lambda i: (0, 0))],
        out_specs=pl.BlockSpec((bm, hdim), lambda i: (i, 0)),
        out_shape=jax.ShapeDtypeStruct((m, hdim), jnp.float32),
    )(parts, b2d)


def kernel(edge_index, edge_weight, emb, W1, b1, W2, b2):
    src = edge_index[0]
    dst = edge_index[1]
    n, d = emb.shape
    zeros = jnp.zeros(((n // NS) & ~7, d), dtype=jnp.float32)

    h1 = _tc_matmul(emb, W1)
    parts1 = _sc_aggregate(h1, src, dst, edge_weight, zeros)
    h2 = _tc_combine_relu_matmul(parts1, b1, W2)
    parts2 = _sc_aggregate(h2, src, dst, edge_weight, zeros)
    return _tc_combine_bias(parts2, b2)


# speedup vs baseline: 10.1941x; 2.4350x over previous
import functools

import jax
import jax.numpy as jnp
from jax import lax
from jax.experimental import pallas as pl
from jax.experimental.pallas import tpu as pltpu
from jax.experimental.pallas import tpu_sc as plsc

NC = 2
NS = 16
NW = NC * NS


def _pick_chunk(epw):
    for c in range(128, 7, -8):
        if epw % c == 0:
            return c
    return None


def _sc_aggregate(h, src, dst, w, zeros):
    n, d = h.shape
    e = src.shape[0]
    epw = e // NW
    ch = _pick_chunk(epw)
    nch = epw // ch
    rpt = (n // NS) & ~7
    tail = n - NS * rpt

    mesh = plsc.VectorSubcoreMesh(core_axis_name="c", subcore_axis_name="s")

    @functools.partial(
        pl.kernel,
        out_type=jax.ShapeDtypeStruct((NC, n, d), jnp.float32),
        mesh=mesh,
        scratch_types=[
            pltpu.VMEM((epw,), jnp.int32),
            pltpu.VMEM((epw,), jnp.int32),
            pltpu.VMEM((epw,), jnp.float32),
            pltpu.VMEM((ch, d), jnp.float32),
            pltpu.VMEM((ch, d), jnp.float32),
            pltpu.VMEM_SHARED((n, d), jnp.float32),
            pltpu.SemaphoreType.DMA,
            pltpu.SemaphoreType.DMA,
        ],
    )
    def k(h_hbm, src_hbm, dst_hbm, w_hbm, z_hbm, out_hbm,
          srcv, dstv, wv, rows_a, rows_b, acc, sem_a, sem_b):
        cid = lax.axis_index("c")
        sid = lax.axis_index("s")
        wid = sid * NC + cid
        woff = pl.multiple_of(wid * epw, 8)
        pltpu.sync_copy(src_hbm.at[pl.ds(woff, epw)], srcv)
        pltpu.sync_copy(dst_hbm.at[pl.ds(woff, epw)], dstv)
        pltpu.sync_copy(w_hbm.at[pl.ds(woff, epw)], wv)
        r0 = pl.multiple_of(sid * rpt, 8)
        pltpu.sync_copy(z_hbm.at[pl.ds(0, rpt)], acc.at[pl.ds(r0, rpt)])
        if tail:
            @pl.when(sid == NS - 1)
            def _():
                pltpu.sync_copy(z_hbm.at[pl.ds(0, tail)],
                                acc.at[pl.ds(NS * rpt, tail)])
        bufs = (rows_a, rows_b)
        sems = (sem_a, sem_b)
        pltpu.async_copy(h_hbm.at[srcv.at[pl.ds(0, ch)]], rows_a, sem_a)
        if nch > 1:
            pltpu.async_copy(h_hbm.at[srcv.at[pl.ds(ch, ch)]], rows_b, sem_b)
        plsc.subcore_barrier()

        def scale_and_scatter(c, rows, sem):
            pltpu.make_async_copy(h_hbm.at[pl.ds(0, ch)], rows, sem).wait()
            base = c * ch
            for g in range(ch // 16):
                wvec = wv[pl.ds(base + g * 16, 16)]
                for j in range(16):
                    ei = g * 16 + j
                    we = wvec[j]
                    for kk in range(d // 16):
                        sl = pl.ds(kk * 16, 16)
                        rows[ei, sl] = rows[ei, sl] * we
            pltpu.sync_copy(rows, acc.at[dstv.at[pl.ds(base, ch)]], add=True)

        def body(i, carry):
            for b in range(2):
                c = 2 * i + b
                scale_and_scatter(c, bufs[b], sems[b])
                nxt = c + 2

                @pl.when(nxt < nch)
                def _():
                    pltpu.async_copy(h_hbm.at[srcv.at[pl.ds(nxt * ch, ch)]],
                                     bufs[b], sems[b])
            return carry

        lax.fori_loop(0, nch // 2, body, 0)
        if nch % 2:
            scale_and_scatter(nch - 1, bufs[(nch - 1) % 2], sems[(nch - 1) % 2])
        plsc.subcore_barrier()
        pltpu.sync_copy(acc.at[pl.ds(r0, rpt)], out_hbm.at[cid, pl.ds(r0, rpt)])
        if tail:
            @pl.when(sid == NS - 1)
            def _():
                pltpu.sync_copy(acc.at[pl.ds(NS * rpt, tail)],
                                out_hbm.at[cid, pl.ds(NS * rpt, tail)])

    return k(h, src, dst, w, zeros)


def _tc_matmul(x, w):
    m, kdim = x.shape
    nout = w.shape[1]
    nb = 10
    bm = m // nb

    def body(x_ref, w_ref, o_ref):
        o_ref[...] = jnp.dot(x_ref[...], w_ref[...],
                             preferred_element_type=jnp.float32)

    return pl.pallas_call(
        body,
        grid=(nb,),
        in_specs=[pl.BlockSpec((bm, kdim), lambda i: (i, 0)),
                  pl.BlockSpec((kdim, nout), lambda i: (0, 0))],
        out_specs=pl.BlockSpec((bm, nout), lambda i: (i, 0)),
        out_shape=jax.ShapeDtypeStruct((m, nout), jnp.float32),
    )(x, w)


def _tc_combine_relu_matmul(parts, b, w):
    _, m, hdim = parts.shape
    nout = w.shape[1]
    nb = 10
    bm = m // nb
    b2d = b.reshape(1, hdim)

    def body(p_ref, b_ref, w_ref, o_ref):
        x = jnp.maximum(p_ref[0] + p_ref[1] + b_ref[...], 0.0)
        o_ref[...] = jnp.dot(x, w_ref[...], preferred_element_type=jnp.float32)

    return pl.pallas_call(
        body,
        grid=(nb,),
        in_specs=[pl.BlockSpec((NC, bm, hdim), lambda i: (0, i, 0)),
                  pl.BlockSpec((1, hdim), lambda i: (0, 0)),
                  pl.BlockSpec((hdim, nout), lambda i: (0, 0))],
        out_specs=pl.BlockSpec((bm, nout), lambda i: (i, 0)),
        out_shape=jax.ShapeDtypeStruct((m, nout), jnp.float32),
    )(parts, b2d, w)


def _tc_combine_bias(parts, b):
    _, m, hdim = parts.shape
    nb = 10
    bm = m // nb
    b2d = b.reshape(1, hdim)

    def body(p_ref, b_ref, o_ref):
        o_ref[...] = p_ref[0] + p_ref[1] + b_ref[...]

    return pl.pallas_call(
        body,
        grid=(nb,),
        in_specs=[pl.BlockSpec((NC, bm, hdim), lambda i: (0, i, 0)),
                  pl.BlockSpec((1, hdim), lambda i: (0, 0))],
        out_specs=pl.BlockSpec((bm, hdim), lambda i: (i, 0)),
        out_shape=jax.ShapeDtypeStruct((m, hdim), jnp.float32),
    )(parts, b2d)


def kernel(edge_index, edge_weight, emb, W1, b1, W2, b2):
    src = edge_index[0]
    dst = edge_index[1]
    n, d = emb.shape
    zeros = jnp.zeros(((n // NS) & ~7, d), dtype=jnp.float32)

    h1 = _tc_matmul(emb, W1)
    parts1 = _sc_aggregate(h1, src, dst, edge_weight, zeros)
    h2 = _tc_combine_relu_matmul(parts1, b1, W2)
    parts2 = _sc_aggregate(h2, src, dst, edge_weight, zeros)
    return _tc_combine_bias(parts2, b2)
